# Initial kernel scaffold; baseline (speedup 1.0000x reference)
import jax
import jax.numpy as jnp
from jax.experimental import pallas as pl


def kernel(x, edge_index, edge_attr, W_enc, b_enc, eps, W1, b1, bn_gamma, bn_beta, W2, b2):
    raise NotImplementedError("write your pallas kernel here")


# speedup vs baseline: 1.4114x; 1.4114x over previous
import functools

import jax
import jax.numpy as jnp
from jax import lax
from jax.experimental import pallas as pl
from jax.experimental.pallas import tpu as pltpu
from jax.experimental.pallas import tpu_sc as plsc

N = 10000
E = 160000
D = 256
DH = 128
NSUB = 16
B = 80
EPT = E // NSUB
NB = EPT // B
RPT = 624
RB = 1000
BN_EPS_C = 1e-5


def _make_sc_aggregate():
    mesh = plsc.VectorSubcoreMesh(core_axis_name="c", subcore_axis_name="s")

    @functools.partial(
        pl.kernel,
        mesh=mesh,
        out_type=jax.ShapeDtypeStruct((2, N, DH), jnp.float32),
        scratch_types=[
            pltpu.VMEM((B,), jnp.int32),
            pltpu.VMEM((B,), jnp.int32),
            pltpu.VMEM((B * 8 + 16,), jnp.float32),
            pltpu.VMEM((B, DH), jnp.float32),
            pltpu.VMEM((B, DH), jnp.float32),
            pltpu.VMEM((7, DH), jnp.float32),
            pltpu.VMEM((DH,), jnp.float32),
            pltpu.VMEM((208, DH), jnp.float32),
            pltpu.VMEM_SHARED((N, DH), jnp.float32),
            pltpu.SemaphoreType.DMA,
        ],
    )
    def sc_aggr(x0, x1, srcs, dsts, attr, wenc, benc,
                out, src_v, dst_v, attr_v, xbuf, mbuf, w_v, b_v, zbuf,
                acc, sem):
        c = lax.axis_index("c")
        s = lax.axis_index("s")
        pltpu.sync_copy(wenc.at[c], w_v)
        pltpu.sync_copy(benc.at[c], b_v)

        z16 = jnp.zeros((16,), jnp.float32)

        def zrow(j, carry):
            for g in range(DH // 16):
                zbuf[j, pl.ds(g * 16, 16)] = z16
            return carry

        lax.fori_loop(0, 208, zrow, 0)
        r0 = pl.multiple_of(s * RPT, 8)
        for t in range(3):
            pltpu.sync_copy(zbuf, acc.at[pl.ds(r0 + t * 208, 208)])

        @pl.when(s == 0)
        def _():
            pltpu.sync_copy(zbuf.at[pl.ds(0, 16)],
                            acc.at[pl.ds(NSUB * RPT, N - NSUB * RPT)])

        plsc.subcore_barrier()

        def batch(bi, carry):
            off = pl.multiple_of(s * EPT + bi * B, 8)
            pltpu.sync_copy(srcs.at[pl.ds(off, B)], src_v)
            pltpu.sync_copy(dsts.at[pl.ds(off, B)], dst_v)
            pltpu.sync_copy(attr.at[pl.ds(off * 8, B * 8)],
                            attr_v.at[pl.ds(0, B * 8)])

            @pl.when(c == 0)
            def _():
                pltpu.async_copy(x0.at[src_v], xbuf, sem).wait()

            @pl.when(c == 1)
            def _():
                pltpu.async_copy(x1.at[src_v], xbuf, sem).wait()

            for p in range(2):
                base = p * 64
                wv = [[w_v[k, pl.ds(base + g * 16, 16)] for g in range(4)]
                      for k in range(7)]
                bv = [b_v[pl.ds(base + g * 16, 16)] for g in range(4)]

                def edge(j, cc):
                    av = attr_v[pl.ds(j * 8, 16)]
                    a = [av[k] for k in range(7)]
                    for g in range(4):
                        e = bv[g]
                        for k in range(7):
                            e = e + a[k] * wv[k][g]
                        xv = xbuf[j, pl.ds(base + g * 16, 16)]
                        mbuf[j, pl.ds(base + g * 16, 16)] = jnp.maximum(
                            xv + e, 0.0)
                    return cc

                lax.fori_loop(0, B, edge, 0)

            pltpu.sync_copy(mbuf, acc.at[dst_v], add=True)
            return carry

        lax.fori_loop(0, NB, batch, 0)
        plsc.subcore_barrier()
        pltpu.sync_copy(acc.at[pl.ds(r0, RPT)], out.at[c, pl.ds(r0, RPT)])

        @pl.when(s == 0)
        def _():
            tail = pl.multiple_of(NSUB * RPT, 8)
            pltpu.sync_copy(acc.at[pl.ds(tail, N - NSUB * RPT)],
                            out.at[c, pl.ds(tail, N - NSUB * RPT)])

    return sc_aggr


_sc_aggregate = _make_sc_aggregate()


def _mlp1_body(eps_ref, x_ref, a0_ref, a1_ref, w1_ref, b1_ref,
               h1_ref, st_ref):
    i = pl.program_id(0)
    scale = 1.0 + eps_ref[0, 0]
    aggr = jnp.concatenate([a0_ref[0], a1_ref[0]], axis=1)
    h = scale * x_ref[...] + aggr
    h1 = jnp.dot(h, w1_ref[...], preferred_element_type=jnp.float32) \
        + b1_ref[...]
    h1_ref[...] = h1

    @pl.when(i == 0)
    def _():
        st_ref[...] = jnp.zeros_like(st_ref)

    upd = jnp.concatenate([
        jnp.sum(h1, axis=0, keepdims=True),
        jnp.sum(h1 * h1, axis=0, keepdims=True),
        jnp.zeros((6, 2 * D), jnp.float32)], axis=0)
    st_ref[...] = st_ref[...] + upd


def _mlp2_body(h1_ref, st_ref, g_ref, be_ref, w2_ref, b2_ref, o_ref):
    mu = st_ref[0:1, :] * (1.0 / N)
    ms = st_ref[1:2, :] * (1.0 / N)
    var = ms - mu * mu
    inv = lax.rsqrt(var + BN_EPS_C)
    h1n = (h1_ref[...] - mu) * (inv * g_ref[...]) + be_ref[...]
    o_ref[...] = jnp.dot(jnp.maximum(h1n, 0.0), w2_ref[...],
                         preferred_element_type=jnp.float32) + b2_ref[...]


def kernel(x, edge_index, edge_attr, W_enc, b_enc, eps,
           W1, b1, bn_gamma, bn_beta, W2, b2):
    src = edge_index[0].astype(jnp.int32)
    dst = edge_index[1].astype(jnp.int32)
    x0 = x[:, :DH]
    x1 = x[:, DH:]
    wenc2 = jnp.stack([W_enc[:, :DH], W_enc[:, DH:]])
    benc2 = jnp.reshape(b_enc, (2, DH))
    attr8 = jnp.reshape(
        jnp.pad(edge_attr, ((0, 0), (0, 1))), (E * 8,))

    aggr2 = _sc_aggregate(x0, x1, src, dst, attr8, wenc2, benc2)

    grid = (N // RB,)
    h1, stats = pl.pallas_call(
        _mlp1_body,
        grid=grid,
        in_specs=[
            pl.BlockSpec((1, 1), lambda i: (0, 0), memory_space=pltpu.SMEM),
            pl.BlockSpec((RB, D), lambda i: (i, 0)),
            pl.BlockSpec((1, RB, DH), lambda i: (0, i, 0)),
            pl.BlockSpec((1, RB, DH), lambda i: (1, i, 0)),
            pl.BlockSpec((D, 2 * D), lambda i: (0, 0)),
            pl.BlockSpec((1, 2 * D), lambda i: (0, 0)),
        ],
        out_specs=[
            pl.BlockSpec((RB, 2 * D), lambda i: (i, 0)),
            pl.BlockSpec((8, 2 * D), lambda i: (0, 0)),
        ],
        out_shape=[
            jax.ShapeDtypeStruct((N, 2 * D), jnp.float32),
            jax.ShapeDtypeStruct((8, 2 * D), jnp.float32),
        ],
    )(jnp.reshape(eps, (1, 1)), x, aggr2, aggr2, W1,
      jnp.reshape(b1, (1, 2 * D)))

    out = pl.pallas_call(
        _mlp2_body,
        grid=grid,
        in_specs=[
            pl.BlockSpec((RB, 2 * D), lambda i: (i, 0)),
            pl.BlockSpec((8, 2 * D), lambda i: (0, 0)),
            pl.BlockSpec((1, 2 * D), lambda i: (0, 0)),
            pl.BlockSpec((1, 2 * D), lambda i: (0, 0)),
            pl.BlockSpec((2 * D, D), lambda i: (0, 0)),
            pl.BlockSpec((1, D), lambda i: (0, 0)),
        ],
        out_specs=pl.BlockSpec((RB, D), lambda i: (i, 0)),
        out_shape=jax.ShapeDtypeStruct((N, D), jnp.float32),
    )(h1, stats, jnp.reshape(bn_gamma, (1, 2 * D)),
      jnp.reshape(bn_beta, (1, 2 * D)), W2, jnp.reshape(b2, (1, D)))

    return out
